# baseline (speedup 1.0000x reference)
import functools

import jax
import jax.numpy as jnp
from jax import lax
from jax.experimental import pallas as pl
from jax.experimental.pallas import tpu as pltpu
from jax.experimental.pallas import tpu_sc as plsc

NN = 10000
NE = 320000
FD = 128
FH = 64

NC = 2
NS = 16
NW = NC * NS
CHUNK = 128
CPT = 80
EPAD = NW * CPT * CHUNK
RPT = 640
NPAD = NS * RPT
CH_TOT = EPAD // CHUNK
C0_CHUNKS = 32
C1_CHUNKS = 128
C0_TOT = NS * C0_CHUNKS

BM = 2000


def _mm2_body(h_ref, wr_ref, wo_ref, b_ref, u_ref, v_ref):
    h = h_ref[...]
    dn = (((1,), (1,)), ((), ()))
    u_ref[...] = lax.dot_general(h, wr_ref[...], dn,
                                 preferred_element_type=jnp.float32)
    v_ref[...] = lax.dot_general(h, wo_ref[...], dn,
                                 preferred_element_type=jnp.float32) + b_ref[...]


def _comb_mm2_body(a0_ref, a1_ref, vp_ref, wr_ref, wo_ref, b_ref, u_ref, v_ref):
    h = jnp.maximum(a0_ref[...] + a1_ref[...] + vp_ref[...], 0.0)
    dn = (((1,), (1,)), ((), ()))
    u_ref[...] = lax.dot_general(h, wr_ref[...], dn,
                                 preferred_element_type=jnp.float32)
    v_ref[...] = lax.dot_general(h, wo_ref[...], dn,
                                 preferred_element_type=jnp.float32) + b_ref[...]


def _final_body(a0_ref, a1_ref, vp_ref, wl_ref, bl_ref, o_ref):
    t = a0_ref[...] + a1_ref[...] + vp_ref[...]
    dn = (((1,), (1,)), ((), ()))
    o_ref[...] = lax.dot_general(t, wl_ref[...], dn,
                                 preferred_element_type=jnp.float32) + bl_ref[...]


_row_spec = pl.BlockSpec((BM, FD), lambda i: (i, 0))
_w_spec = pl.BlockSpec((FD, FD), lambda i: (0, 0))
_b_spec = pl.BlockSpec((1, FD), lambda i: (0, 0))
_uv_shape = [jax.ShapeDtypeStruct((NN, FD), jnp.float32)] * 2

_mm2 = pl.pallas_call(
    _mm2_body,
    grid=(NN // BM,),
    in_specs=[_row_spec, _w_spec, _w_spec, _b_spec],
    out_specs=[_row_spec, _row_spec],
    out_shape=_uv_shape,
)

_comb_mm2 = pl.pallas_call(
    _comb_mm2_body,
    grid=(NN // BM,),
    in_specs=[_row_spec, _row_spec, _row_spec, _w_spec, _w_spec, _b_spec],
    out_specs=[_row_spec, _row_spec],
    out_shape=_uv_shape,
)

_final = pl.pallas_call(
    _final_body,
    grid=(NN // BM,),
    in_specs=[_row_spec, _row_spec, _row_spec, _w_spec, _b_spec],
    out_specs=_row_spec,
    out_shape=jax.ShapeDtypeStruct((NN, FD), jnp.float32),
)


_mesh = plsc.VectorSubcoreMesh(core_axis_name="c", subcore_axis_name="s")


PIECE = 32

@functools.partial(
    pl.kernel,
    mesh=_mesh,
    out_type=jax.ShapeDtypeStruct((NC, NPAD, FD), jnp.float32),
    scratch_types=[
        pltpu.VMEM((PIECE, CHUNK), jnp.int32),
        pltpu.VMEM((PIECE, CHUNK), jnp.int32),
        pltpu.VMEM((CHUNK, FD), jnp.float32),
        pltpu.VMEM((CHUNK, FD), jnp.float32),
        pltpu.VMEM_SHARED((NPAD, FD), jnp.float32),
        pltpu.SemaphoreType.DMA,
        pltpu.SemaphoreType.DMA,
    ],
)
def _sc_agg(src_hbm, dst_hbm, u_hbm, z_hbm, out_hbm,
            src_v, dst_v, rows0, rows1, acc_sh, sem0, sem1):
    c = lax.axis_index("c")
    s = lax.axis_index("s")
    rslc = pl.ds(s * RPT, RPT)

    pltpu.sync_copy(z_hbm, acc_sh.at[rslc])
    plsc.subcore_barrier()

    n_pieces = jnp.where(c == 0, C0_CHUNKS // PIECE, C1_CHUNKS // PIECE)
    tile_base = jnp.where(c == 0, s * C0_CHUNKS, C0_TOT + s * C1_CHUNKS)

    def piece_body(p, carry):
        cbase = tile_base + p * PIECE
        pltpu.sync_copy(src_hbm.at[pl.ds(cbase, PIECE)], src_v)
        pltpu.sync_copy(dst_hbm.at[pl.ds(cbase, PIECE)], dst_v)
        pltpu.async_copy(u_hbm.at[src_v.at[0]], rows0, sem0)

        def pair(i, inner):
            j = 2 * i
            pltpu.async_copy(u_hbm.at[src_v.at[j + 1]], rows1, sem1)
            pltpu.make_async_copy(u_hbm.at[src_v.at[j]], rows0, sem0).wait()
            pltpu.sync_copy(rows0, acc_sh.at[dst_v.at[j]], add=True)

            @pl.when(j + 2 < PIECE)
            def _():
                pltpu.async_copy(u_hbm.at[src_v.at[j + 2]], rows0, sem0)

            pltpu.make_async_copy(u_hbm.at[src_v.at[j + 1]], rows1, sem1).wait()
            pltpu.sync_copy(rows1, acc_sh.at[dst_v.at[j + 1]], add=True)
            return inner

        lax.fori_loop(0, PIECE // 2, pair, 0)
        return carry

    lax.fori_loop(0, n_pieces, piece_body, 0)

    plsc.subcore_barrier()
    pltpu.sync_copy(acc_sh.at[rslc], out_hbm.at[c].at[rslc])


def kernel(x, edge_index, W1_rel, b1_rel, W1_root, W2_rel, b2_rel, W2_root,
           W3_rel, b3_rel, W3_root, W_lin, b_lin):
    pad = EPAD - NE
    src_p = jnp.concatenate(
        [edge_index[0], jnp.zeros((pad,), jnp.int32)]).reshape(CH_TOT, CHUNK)
    dst_p = jnp.concatenate(
        [edge_index[1], jnp.full((pad,), NN, jnp.int32)]).reshape(CH_TOT, CHUNK)
    zrows = jnp.zeros((RPT, FD), jnp.float32)

    u, v = _mm2(x, W1_rel, W1_root, b1_rel.reshape(1, FD))
    agg = _sc_agg(src_p, dst_p, u, zrows)
    u, v = _comb_mm2(agg[0], agg[1], v, W2_rel, W2_root, b2_rel.reshape(1, FD))
    agg = _sc_agg(src_p, dst_p, u, zrows)
    u, v = _comb_mm2(agg[0], agg[1], v, W3_rel, W3_root, b3_rel.reshape(1, FD))
    agg = _sc_agg(src_p, dst_p, u, zrows)
    return _final(agg[0], agg[1], v, W_lin, b_lin.reshape(1, FD))


# speedup vs baseline: 1.1627x; 1.1627x over previous
import functools

import jax
import jax.numpy as jnp
from jax import lax
from jax.experimental import pallas as pl
from jax.experimental.pallas import tpu as pltpu
from jax.experimental.pallas import tpu_sc as plsc

NN = 10000
NE = 320000
FD = 128
FH = 64

NC = 2
NS = 16
NW = NC * NS
CHUNK = 128
CPT = 80
EPAD = NW * CPT * CHUNK
RPT = 640
NPAD = NS * RPT
CH_TOT = EPAD // CHUNK
C0_CHUNKS = 128
C1_CHUNKS = 32
C0_TOT = NS * C0_CHUNKS

BM = 2000


def _mm2_body(h_ref, wr_ref, wo_ref, b_ref, u_ref, v_ref):
    h = h_ref[...]
    dn = (((1,), (1,)), ((), ()))
    u_ref[...] = lax.dot_general(h, wr_ref[...], dn,
                                 preferred_element_type=jnp.float32)
    v_ref[...] = lax.dot_general(h, wo_ref[...], dn,
                                 preferred_element_type=jnp.float32) + b_ref[...]


def _comb_mm2_body(a0_ref, a1_ref, vp_ref, wr_ref, wo_ref, b_ref, u_ref, v_ref):
    h = jnp.maximum(a0_ref[...] + a1_ref[...] + vp_ref[...], 0.0)
    dn = (((1,), (1,)), ((), ()))
    u_ref[...] = lax.dot_general(h, wr_ref[...], dn,
                                 preferred_element_type=jnp.float32)
    v_ref[...] = lax.dot_general(h, wo_ref[...], dn,
                                 preferred_element_type=jnp.float32) + b_ref[...]


def _final_body(a0_ref, a1_ref, vp_ref, wl_ref, bl_ref, o_ref):
    t = a0_ref[...] + a1_ref[...] + vp_ref[...]
    dn = (((1,), (1,)), ((), ()))
    o_ref[...] = lax.dot_general(t, wl_ref[...], dn,
                                 preferred_element_type=jnp.float32) + bl_ref[...]


_row_spec = pl.BlockSpec((BM, FD), lambda i: (i, 0))
_w_spec = pl.BlockSpec((FD, FD), lambda i: (0, 0))
_b_spec = pl.BlockSpec((1, FD), lambda i: (0, 0))
_uv_shape = [jax.ShapeDtypeStruct((NN, FD), jnp.float32)] * 2

_mm2 = pl.pallas_call(
    _mm2_body,
    grid=(NN // BM,),
    in_specs=[_row_spec, _w_spec, _w_spec, _b_spec],
    out_specs=[_row_spec, _row_spec],
    out_shape=_uv_shape,
)

_comb_mm2 = pl.pallas_call(
    _comb_mm2_body,
    grid=(NN // BM,),
    in_specs=[_row_spec, _row_spec, _row_spec, _w_spec, _w_spec, _b_spec],
    out_specs=[_row_spec, _row_spec],
    out_shape=_uv_shape,
)

_final = pl.pallas_call(
    _final_body,
    grid=(NN // BM,),
    in_specs=[_row_spec, _row_spec, _row_spec, _w_spec, _b_spec],
    out_specs=_row_spec,
    out_shape=jax.ShapeDtypeStruct((NN, FD), jnp.float32),
)


_mesh = plsc.VectorSubcoreMesh(core_axis_name="c", subcore_axis_name="s")


PIECE = 32

@functools.partial(
    pl.kernel,
    mesh=_mesh,
    out_type=jax.ShapeDtypeStruct((NC, NPAD, FD), jnp.float32),
    scratch_types=[
        pltpu.VMEM((PIECE, CHUNK), jnp.int32),
        pltpu.VMEM((PIECE, CHUNK), jnp.int32),
        pltpu.VMEM((CHUNK, FD), jnp.float32),
        pltpu.VMEM((CHUNK, FD), jnp.float32),
        pltpu.VMEM_SHARED((NPAD, FD), jnp.float32),
        pltpu.SemaphoreType.DMA,
        pltpu.SemaphoreType.DMA,
    ],
)
def _sc_agg(src_hbm, dst_hbm, u_hbm, z_hbm, out_hbm,
            src_v, dst_v, rows0, rows1, acc_sh, sem0, sem1):
    c = lax.axis_index("c")
    s = lax.axis_index("s")
    rslc = pl.ds(s * RPT, RPT)

    pltpu.sync_copy(z_hbm, acc_sh.at[rslc])
    plsc.subcore_barrier()

    n_pieces = jnp.where(c == 0, C0_CHUNKS // PIECE, C1_CHUNKS // PIECE)
    tile_base = jnp.where(c == 0, s * C0_CHUNKS, C0_TOT + s * C1_CHUNKS)

    def piece_body(p, carry):
        cbase = tile_base + p * PIECE
        pltpu.sync_copy(src_hbm.at[pl.ds(cbase, PIECE)], src_v)
        pltpu.sync_copy(dst_hbm.at[pl.ds(cbase, PIECE)], dst_v)
        pltpu.async_copy(u_hbm.at[src_v.at[0]], rows0, sem0)

        def pair(i, inner):
            j = 2 * i
            pltpu.async_copy(u_hbm.at[src_v.at[j + 1]], rows1, sem1)
            pltpu.make_async_copy(u_hbm.at[src_v.at[j]], rows0, sem0).wait()
            pltpu.sync_copy(rows0, acc_sh.at[dst_v.at[j]], add=True)

            @pl.when(j + 2 < PIECE)
            def _():
                pltpu.async_copy(u_hbm.at[src_v.at[j + 2]], rows0, sem0)

            pltpu.make_async_copy(u_hbm.at[src_v.at[j + 1]], rows1, sem1).wait()
            pltpu.sync_copy(rows1, acc_sh.at[dst_v.at[j + 1]], add=True)
            return inner

        lax.fori_loop(0, PIECE // 2, pair, 0)
        return carry

    lax.fori_loop(0, n_pieces, piece_body, 0)

    plsc.subcore_barrier()
    pltpu.sync_copy(acc_sh.at[rslc], out_hbm.at[c].at[rslc])


def kernel(x, edge_index, W1_rel, b1_rel, W1_root, W2_rel, b2_rel, W2_root,
           W3_rel, b3_rel, W3_root, W_lin, b_lin):
    pad = EPAD - NE
    src_p = jnp.concatenate(
        [edge_index[0], jnp.zeros((pad,), jnp.int32)]).reshape(CH_TOT, CHUNK)
    dst_p = jnp.concatenate(
        [edge_index[1], jnp.full((pad,), NN, jnp.int32)]).reshape(CH_TOT, CHUNK)
    zrows = jnp.zeros((RPT, FD), jnp.float32)

    u, v = _mm2(x, W1_rel, W1_root, b1_rel.reshape(1, FD))
    agg = _sc_agg(src_p, dst_p, u, zrows)
    u, v = _comb_mm2(agg[0], agg[1], v, W2_rel, W2_root, b2_rel.reshape(1, FD))
    agg = _sc_agg(src_p, dst_p, u, zrows)
    u, v = _comb_mm2(agg[0], agg[1], v, W3_rel, W3_root, b3_rel.reshape(1, FD))
    agg = _sc_agg(src_p, dst_p, u, zrows)
    return _final(agg[0], agg[1], v, W_lin, b_lin.reshape(1, FD))
